# baseline (speedup 1.0000x reference)
import functools

import jax
import jax.numpy as jnp
from jax import lax
from jax.experimental import pallas as pl
from jax.experimental.pallas import tpu as pltpu
from jax.experimental.pallas import tpu_sc as plsc

N_NODES = 10000
N_EDGES = 320000
NC = 2
NS = 16
NW = NC * NS
CHUNK = 128
EPW = 10240
EPAD = EPW * NW
NCHUNK = EPW // CHUNK
ACC_ROWS = 10240
RPT = ACC_ROWS // NS


def _make_sc_aggregate(d, with_cnt):
  mesh = plsc.VectorSubcoreMesh(
      core_axis_name="c", subcore_axis_name="s",
      num_cores=NC, num_subcores=NS)

  kb = 1 if d == 128 else 4
  nsup = NCHUNK // kb
  assert (nsup - 2) % 6 == 0

  out_type = [jax.ShapeDtypeStruct((NC, ACC_ROWS, d), jnp.float32)]
  scratch = [
      pltpu.VMEM((3, kb, CHUNK), jnp.int32),
      pltpu.VMEM((3, kb, CHUNK), jnp.int32),
      pltpu.VMEM((2, kb * CHUNK, d), jnp.float32),
      pltpu.VMEM_SHARED((ACC_ROWS, d), jnp.float32),
      pltpu.SemaphoreType.DMA,
      pltpu.SemaphoreType.DMA,
      pltpu.SemaphoreType.DMA,
  ]
  if with_cnt:
    out_type.append(jax.ShapeDtypeStruct((NC, ACC_ROWS), jnp.float32))
    scratch += [
        pltpu.VMEM((CHUNK,), jnp.float32),
        pltpu.VMEM_SHARED((ACC_ROWS,), jnp.float32),
        pltpu.SemaphoreType.DMA,
    ]

  @functools.partial(
      pl.kernel, out_type=out_type, mesh=mesh, scratch_types=scratch,
      compiler_params=pltpu.CompilerParams(use_tc_tiling_on_sc=False))
  def agg_kernel(x_hbm, src_hbm, dst_hbm, z2d_hbm, z1d_hbm, ones_hbm, *rest):
    if with_cnt:
      (acc_out, cnt_out, srcv, dstv, rows, acc_s, gsem, isem, ssem, ones,
       cnt_s, csem) = rest
    else:
      acc_out, srcv, dstv, rows, acc_s, gsem, isem, ssem = rest
    c = lax.axis_index("c")
    s = lax.axis_index("s")
    w = c * NS + s
    rbase = pl.multiple_of(s * RPT, 8)

    pltpu.sync_copy(z2d_hbm, acc_s.at[pl.ds(rbase, RPT)])
    if with_cnt:
      pltpu.sync_copy(z1d_hbm, cnt_s.at[pl.ds(rbase, RPT)])
      pltpu.sync_copy(ones_hbm, ones)
    plsc.subcore_barrier()

    wrow = w * NCHUNK

    def idx_desc(sup, islot):
      row = wrow + sup * kb
      return [
          pltpu.make_async_copy(src_hbm.at[pl.ds(row, kb)], srcv.at[islot],
                                isem),
          pltpu.make_async_copy(dst_hbm.at[pl.ds(row, kb)], dstv.at[islot],
                                isem),
      ]

    def idx_start(sup, islot):
      for dsc in idx_desc(sup, islot):
        dsc.start()

    def idx_wait(sup, islot):
      for dsc in idx_desc(sup, islot):
        dsc.wait()

    def gath_desc(islot, slot):
      return [pltpu.make_async_copy(
          x_hbm.at[srcv.at[islot, b]],
          rows.at[slot, pl.ds(b * CHUNK, CHUNK)], gsem) for b in range(kb)]

    def gath_start(islot, slot):
      for dsc in gath_desc(islot, slot):
        dsc.start()

    def gath_wait(islot, slot):
      for dsc in gath_desc(islot, slot):
        dsc.wait()

    def scat_desc(islot, slot):
      out = []
      for b in range(kb):
        out.append(pltpu.make_async_copy(
            rows.at[slot, pl.ds(b * CHUNK, CHUNK)],
            acc_s.at[dstv.at[islot, b]], ssem))
        if with_cnt:
          out.append(pltpu.make_async_copy(
              ones, cnt_s.at[dstv.at[islot, b]], csem))
      return out

    def scat_start(islot, slot):
      for dsc in scat_desc(islot, slot):
        dsc.start(add=True)

    def scat_wait(islot, slot):
      for dsc in scat_desc(islot, slot):
        dsc.wait()

    idx_start(0, 0)
    idx_start(1, 1)
    idx_wait(0, 0)
    gath_start(0, 0)
    idx_start(2, 2)
    idx_wait(1, 1)
    gath_start(1, 1)
    gath_wait(0, 0)
    scat_start(0, 0)

    def stage(i, k):
      scat_wait((k + 2) % 3, (k + 1) % 2)
      idx_start(i + 2, (k + 2) % 3)
      idx_wait(i + 1, (k + 1) % 3)
      gath_start((k + 1) % 3, (k + 1) % 2)
      gath_wait(k % 3, k % 2)
      scat_start(k % 3, k % 2)

    def body(t, carry):
      i0 = t * 6 + 1
      for k in range(6):
        stage(i0 + k, (1 + k) % 6)
      return carry

    lax.fori_loop(0, (nsup - 2) // 6, body, 0)
    idx_wait(nsup, 2)
    scat_wait(0, 0)
    gath_wait(1, 1)
    scat_start(1, 1)
    scat_wait(1, 1)
    plsc.subcore_barrier()

    pltpu.sync_copy(acc_s.at[pl.ds(rbase, RPT)],
                    acc_out.at[c, pl.ds(rbase, RPT)])
    if with_cnt:
      pltpu.sync_copy(cnt_s.at[pl.ds(rbase, RPT)],
                      cnt_out.at[c, pl.ds(rbase, RPT)])

  return agg_kernel


_get_sc_aggregate = functools.lru_cache(maxsize=None)(_make_sc_aggregate)

BLK = 1000


def _tc1_body(a0, a1, c0, c1, xr, wl, bl, wr, w2l, w2r, hp, hq):
  cnt = c0[...] + c1[...]
  inv = 1.0 / jnp.maximum(cnt, 1.0)
  mean = (a0[...] + a1[...]) * inv
  h = jnp.maximum(
      jnp.dot(mean, wl[...], preferred_element_type=jnp.float32) + bl[...]
      + jnp.dot(xr[...], wr[...], preferred_element_type=jnp.float32), 0.0)
  hp[...] = jnp.dot(h, w2l[...], preferred_element_type=jnp.float32)
  hq[...] = jnp.dot(h, w2r[...], preferred_element_type=jnp.float32)


_tc1 = pl.pallas_call(
    _tc1_body,
    grid=(N_NODES // BLK,),
    in_specs=[
        pl.BlockSpec((BLK, 128), lambda i: (i, 0)),
        pl.BlockSpec((BLK, 128), lambda i: (i, 0)),
        pl.BlockSpec((BLK, 1), lambda i: (i, 0)),
        pl.BlockSpec((BLK, 1), lambda i: (i, 0)),
        pl.BlockSpec((BLK, 128), lambda i: (i, 0)),
        pl.BlockSpec((128, 128), lambda i: (0, 0)),
        pl.BlockSpec((1, 128), lambda i: (0, 0)),
        pl.BlockSpec((128, 128), lambda i: (0, 0)),
        pl.BlockSpec((128, 64), lambda i: (0, 0)),
        pl.BlockSpec((128, 64), lambda i: (0, 0)),
    ],
    out_specs=[
        pl.BlockSpec((BLK, 64), lambda i: (i, 0)),
        pl.BlockSpec((BLK, 64), lambda i: (i, 0)),
    ],
    out_shape=[jax.ShapeDtypeStruct((N_NODES, 64), jnp.float32)] * 2,
)


def _tc2_body(a0, a1, c0, c1, hqr, bl, out):
  cnt = c0[...] + c1[...]
  inv = 1.0 / jnp.maximum(cnt, 1.0)
  out[...] = jnp.maximum((a0[...] + a1[...]) * inv + bl[...] + hqr[...], 0.0)


_tc2 = pl.pallas_call(
    _tc2_body,
    grid=(N_NODES // BLK,),
    in_specs=[
        pl.BlockSpec((BLK, 64), lambda i: (i, 0)),
        pl.BlockSpec((BLK, 64), lambda i: (i, 0)),
        pl.BlockSpec((BLK, 1), lambda i: (i, 0)),
        pl.BlockSpec((BLK, 1), lambda i: (i, 0)),
        pl.BlockSpec((BLK, 64), lambda i: (i, 0)),
        pl.BlockSpec((1, 64), lambda i: (0, 0)),
    ],
    out_specs=pl.BlockSpec((BLK, 64), lambda i: (i, 0)),
    out_shape=jax.ShapeDtypeStruct((N_NODES, 64), jnp.float32),
)


@jax.jit
def kernel(x, edge_index, W1l, b1l, W1r, W2l, b2l, W2r):
  src = edge_index[0]
  dst = edge_index[1]
  pad = (NW * NCHUNK + 8) * CHUNK - N_EDGES
  srcp = jnp.concatenate([src, jnp.zeros((pad,), jnp.int32)])
  srcp = srcp.reshape(NW * NCHUNK + 8, CHUNK)
  dstp = jnp.concatenate([dst, jnp.full((pad,), N_NODES, jnp.int32)])
  dstp = dstp.reshape(NW * NCHUNK + 8, CHUNK)
  z2d128 = jnp.zeros((RPT, 128), jnp.float32)
  z2d64 = jnp.zeros((RPT, 64), jnp.float32)
  z1d = jnp.zeros((RPT,), jnp.float32)
  ones1 = jnp.ones((CHUNK,), jnp.float32)

  aggp1, cntp = _get_sc_aggregate(128, True)(x, srcp, dstp, z2d128, z1d, ones1)
  cnt0 = cntp[0, :N_NODES, None]
  cnt1 = cntp[1, :N_NODES, None]
  hp, hq = _tc1(aggp1[0, :N_NODES], aggp1[1, :N_NODES], cnt0, cnt1, x,
                W1l.T, b1l[None, :], W1r.T, W2l.T, W2r.T)

  res2 = _get_sc_aggregate(64, False)(hp, srcp, dstp, z2d64, z1d, ones1)
  aggp2 = res2[0] if isinstance(res2, (list, tuple)) else res2
  out = _tc2(aggp2[0, :N_NODES], aggp2[1, :N_NODES], cnt0, cnt1, hq,
             b2l[None, :])
  return out
